# Initial kernel scaffold; baseline (speedup 1.0000x reference)
import jax
import jax.numpy as jnp
from jax.experimental import pallas as pl


def kernel(edge_attr, W0, W1, W2):
    raise NotImplementedError("write your pallas kernel here")


# speedup vs baseline: 3.6889x; 3.6889x over previous
import functools

import jax
import jax.numpy as jnp
from jax import lax
from jax.experimental import pallas as pl
from jax.experimental.pallas import tpu as pltpu
from jax.experimental.pallas import tpu_sc as plsc

E = 320000
H = 128
V = 6
NC = 2
NS = 16
NW = NC * NS
BPW = E // NW
CHUNK = 80
NCHUNK = BPW // CHUNK
GRP = CHUNK // 16


def _table_body(w0, w1, w2, out):
    t12 = (w1[...][:, None, :] + w2[...][None, :, :]).reshape(V * V, H)
    out[...] = (w0[...][:, None, :] + t12[None, :, :]).reshape(V * V * V, H)


def _build_table(W0, W1, W2):
    return pl.pallas_call(
        _table_body,
        out_shape=jax.ShapeDtypeStruct((V * V * V, H), jnp.float32),
    )(W0, W1, W2)


def _sc_body(ea_hbm, tbl_hbm, out_hbm, ea_v, idx_v, rows_v, sem):
    wid = lax.axis_index("s") * NC + lax.axis_index("c")
    ebase = wid * BPW

    pltpu.sync_copy(ea_hbm.at[pl.ds(ebase * 3, BPW * 3)], ea_v)

    lane = lax.iota(jnp.int32, 16)

    def idx_group(g, _):
        base3 = g * 48
        a0 = plsc.load_gather(ea_v, [base3 + lane * 3])
        a1 = plsc.load_gather(ea_v, [base3 + lane * 3 + 1])
        a2 = plsc.load_gather(ea_v, [base3 + lane * 3 + 2])
        idx_v[g // GRP, pl.ds((g % GRP) * 16, 16)] = a0 * 36 + a1 * 6 + a2
        return 0

    lax.fori_loop(0, NCHUNK * GRP, idx_group, 0)

    def chunk(k, _):
        pltpu.async_copy(tbl_hbm.at[idx_v.at[k]], rows_v, sem).wait()
        pltpu.sync_copy(rows_v, out_hbm.at[pl.ds(ebase + k * CHUNK, CHUNK)])
        return 0

    lax.fori_loop(0, NCHUNK, chunk, 0)


@functools.partial(jax.jit, donate_argnums=())
def kernel(edge_attr, W0, W1, W2):
    tbl = _build_table(W0, W1, W2)
    ea_flat = edge_attr.astype(jnp.int32).reshape(E * 3)

    sc = pl.kernel(
        _sc_body,
        out_type=jax.ShapeDtypeStruct((E, H), jnp.float32),
        mesh=plsc.VectorSubcoreMesh(core_axis_name="c", subcore_axis_name="s"),
        compiler_params=pltpu.CompilerParams(needs_layout_passes=False),
        scratch_types=[
            pltpu.VMEM((BPW * 3,), jnp.int32),
            pltpu.VMEM((NCHUNK, CHUNK), jnp.int32),
            pltpu.VMEM((CHUNK, H), jnp.float32),
            pltpu.SemaphoreType.DMA,
        ],
    )
    return sc(ea_flat, tbl)


# speedup vs baseline: 3.7203x; 1.0085x over previous
import functools

import jax
import jax.numpy as jnp
from jax import lax
from jax.experimental import pallas as pl
from jax.experimental.pallas import tpu as pltpu
from jax.experimental.pallas import tpu_sc as plsc

E = 320000
H = 128
V = 6
NT = V * V * V
NC = 2
NS = 16
NW = NC * NS
BPW = E // NW
CHUNK = 80
NCHUNK = BPW // CHUNK
GRP = CHUNK // 16
NPAIR = (NCHUNK - 1) // 2


def _table_body(w0, w1, w2, out):
    t12 = (w1[...][:, None, :] + w2[...][None, :, :]).reshape(V * V, H)
    out[...] = (w0[...][:, None, :] + t12[None, :, :]).reshape(NT, H)


def _build_table(W0, W1, W2):
    return pl.pallas_call(
        _table_body,
        out_shape=jax.ShapeDtypeStruct((NT, H), jnp.float32),
    )(W0, W1, W2)


def _sc_body(ea_hbm, tbl_hbm, out_hbm, ea_v, idx_v, tbl_v, r0, r1, sem, o0, o1):
    wid = lax.axis_index("s") * NC + lax.axis_index("c")
    ebase = wid * BPW

    pltpu.async_copy(ea_hbm.at[pl.ds(ebase * 3, BPW * 3)], ea_v, sem)
    pltpu.sync_copy(tbl_hbm, tbl_v)
    pltpu.make_async_copy(ea_hbm.at[pl.ds(0, BPW * 3)], ea_v, sem).wait()

    lane = lax.iota(jnp.int32, 16)

    def idx_group(g, _):
        base3 = g * 48
        a0 = plsc.load_gather(ea_v, [base3 + lane * 3])
        a1 = plsc.load_gather(ea_v, [base3 + lane * 3 + 1])
        a2 = plsc.load_gather(ea_v, [base3 + lane * 3 + 2])
        idx_v[g // GRP, pl.ds((g % GRP) * 16, 16)] = a0 * 36 + a1 * 6 + a2
        return 0

    lax.fori_loop(0, NCHUNK * GRP, idx_group, 0)

    def expand(k, rv):
        def group(g, _):
            idx16 = idx_v[k, pl.ds(g * 16, 16)]
            e0 = g * 16
            for j in range(16):
                s = idx16[j]
                for c in range(H // 16):
                    rv[e0 + j, pl.ds(c * 16, 16)] = tbl_v[s, pl.ds(c * 16, 16)]
            return 0

        lax.fori_loop(0, GRP, group, 0)

    def out_start(k, rv, osem):
        return pltpu.async_copy(rv, out_hbm.at[pl.ds(ebase + k * CHUNK, CHUNK)], osem)

    def out_wait(rv, osem):
        pltpu.make_async_copy(rv, out_hbm.at[pl.ds(0, CHUNK)], osem).wait()

    expand(0, r0)
    out_start(0, r0, o0)

    def pair(i, _):
        a = 2 * i + 1
        b = 2 * i + 2

        @pl.when(i > 0)
        def _():
            out_wait(r1, o1)

        expand(a, r1)
        out_start(a, r1, o1)

        out_wait(r0, o0)
        expand(b, r0)
        out_start(b, r0, o0)
        return 0

    lax.fori_loop(0, NPAIR, pair, 0)
    out_wait(r1, o1)
    out_wait(r0, o0)


@functools.partial(jax.jit, donate_argnums=())
def kernel(edge_attr, W0, W1, W2):
    tbl = _build_table(W0, W1, W2)
    ea_flat = edge_attr.astype(jnp.int32).reshape(E * 3)

    sc = pl.kernel(
        _sc_body,
        out_type=jax.ShapeDtypeStruct((E, H), jnp.float32),
        mesh=plsc.VectorSubcoreMesh(core_axis_name="c", subcore_axis_name="s"),
        compiler_params=pltpu.CompilerParams(needs_layout_passes=False),
        scratch_types=[
            pltpu.VMEM((BPW * 3,), jnp.int32),
            pltpu.VMEM((NCHUNK, CHUNK), jnp.int32),
            pltpu.VMEM((NT, H), jnp.float32),
            pltpu.VMEM((CHUNK, H), jnp.float32),
            pltpu.VMEM((CHUNK, H), jnp.float32),
            pltpu.SemaphoreType.DMA,
            pltpu.SemaphoreType.DMA,
            pltpu.SemaphoreType.DMA,
        ],
    )
    return sc(ea_flat, tbl)


# speedup vs baseline: 6.1476x; 1.6524x over previous
import functools

import jax
import jax.numpy as jnp
from jax import lax
from jax.experimental import pallas as pl
from jax.experimental.pallas import tpu as pltpu
from jax.experimental.pallas import tpu_sc as plsc

E = 320000
H = 128
HG = H // 16
V = 6
NT = V * V * V
NC = 2
NS = 16
NW = NC * NS
BPW = E // NW
CHUNK = 80
NCHUNK = BPW // CHUNK
GRP = CHUNK // 16
NPAIR = (NCHUNK - 1) // 2


def _sc_body(ea_hbm, w0_hbm, w1_hbm, w2_hbm, out_hbm,
             ea_v, idx_v, w0_v, w1_v, w2_v, t12_v, tbl_v, r0, r1, sem, o0, o1):
    wid = lax.axis_index("s") * NC + lax.axis_index("c")
    ebase = wid * BPW

    pltpu.async_copy(ea_hbm.at[pl.ds(ebase * 3, BPW * 3)], ea_v, sem)
    pltpu.sync_copy(w0_hbm, w0_v)
    pltpu.sync_copy(w1_hbm, w1_v)
    pltpu.sync_copy(w2_hbm, w2_v)

    def t12_row(r, _):
        j = r // V
        k = r % V
        for c in range(HG):
            t12_v[r, pl.ds(c * 16, 16)] = (
                w1_v[j, pl.ds(c * 16, 16)] + w2_v[k, pl.ds(c * 16, 16)]
            )
        return 0

    lax.fori_loop(0, V * V, t12_row, 0)

    def tbl_row(r, _):
        i = r // (V * V)
        m = r % (V * V)
        for c in range(HG):
            tbl_v[r, pl.ds(c * 16, 16)] = (
                w0_v[i, pl.ds(c * 16, 16)] + t12_v[m, pl.ds(c * 16, 16)]
            )
        return 0

    lax.fori_loop(0, NT, tbl_row, 0)

    pltpu.make_async_copy(ea_hbm.at[pl.ds(0, BPW * 3)], ea_v, sem).wait()
    lane = lax.iota(jnp.int32, 16)

    def idx_group(g, _):
        base3 = g * 48
        a0 = plsc.load_gather(ea_v, [base3 + lane * 3])
        a1 = plsc.load_gather(ea_v, [base3 + lane * 3 + 1])
        a2 = plsc.load_gather(ea_v, [base3 + lane * 3 + 2])
        idx_v[g // GRP, pl.ds((g % GRP) * 16, 16)] = a0 * 36 + a1 * 6 + a2
        return 0

    lax.fori_loop(0, NCHUNK * GRP, idx_group, 0)

    def expand(k, rv):
        def group(g, _):
            idx16 = idx_v[k, pl.ds(g * 16, 16)]
            e0 = g * 16
            for j in range(16):
                s = idx16[j]
                row = [tbl_v[s, pl.ds(c * 16, 16)] for c in range(HG)]
                for c in range(HG):
                    rv[e0 + j, pl.ds(c * 16, 16)] = row[c]
            return 0

        lax.fori_loop(0, GRP, group, 0)

    def out_start(k, rv, osem):
        return pltpu.async_copy(rv, out_hbm.at[pl.ds(ebase + k * CHUNK, CHUNK)], osem)

    def out_wait(rv, osem):
        pltpu.make_async_copy(rv, out_hbm.at[pl.ds(0, CHUNK)], osem).wait()

    expand(0, r0)
    out_start(0, r0, o0)

    def pair(i, _):
        a = 2 * i + 1
        b = 2 * i + 2

        @pl.when(i > 0)
        def _():
            out_wait(r1, o1)

        expand(a, r1)
        out_start(a, r1, o1)

        out_wait(r0, o0)
        expand(b, r0)
        out_start(b, r0, o0)
        return 0

    lax.fori_loop(0, NPAIR, pair, 0)
    out_wait(r1, o1)
    out_wait(r0, o0)


@functools.partial(jax.jit, donate_argnums=())
def kernel(edge_attr, W0, W1, W2):
    ea_flat = edge_attr.astype(jnp.int32).reshape(E * 3)

    sc = pl.kernel(
        _sc_body,
        out_type=jax.ShapeDtypeStruct((E, H), jnp.float32),
        mesh=plsc.VectorSubcoreMesh(core_axis_name="c", subcore_axis_name="s"),
        compiler_params=pltpu.CompilerParams(needs_layout_passes=False),
        scratch_types=[
            pltpu.VMEM((BPW * 3,), jnp.int32),
            pltpu.VMEM((NCHUNK, CHUNK), jnp.int32),
            pltpu.VMEM((V, H), jnp.float32),
            pltpu.VMEM((V, H), jnp.float32),
            pltpu.VMEM((V, H), jnp.float32),
            pltpu.VMEM((V * V, H), jnp.float32),
            pltpu.VMEM((NT, H), jnp.float32),
            pltpu.VMEM((CHUNK, H), jnp.float32),
            pltpu.VMEM((CHUNK, H), jnp.float32),
            pltpu.SemaphoreType.DMA,
            pltpu.SemaphoreType.DMA,
            pltpu.SemaphoreType.DMA,
        ],
    )
    return sc(ea_flat, W0, W1, W2)


# speedup vs baseline: 13.0607x; 2.1245x over previous
import functools

import jax
import jax.numpy as jnp
from jax import lax
from jax.experimental import pallas as pl
from jax.experimental.pallas import tpu as pltpu
from jax.experimental.pallas import tpu_sc as plsc

E = 320000
H = 128
HG = H // 16
V = 6
NT = V * V * V
NC = 2
NS = 16
NW = NC * NS
BPW = E // NW
CHUNK = 80
NCHUNK = BPW // CHUNK
GRP = CHUNK // 16
NBUF = 4


def _sc_body(a0_hbm, a1_hbm, a2_hbm, w0_hbm, w1_hbm, w2_hbm, out_hbm,
             a0_v, a1_v, a2_v, idx_v, w0_v, w1_v, w2_v, t12_v, tbl_v,
             r0, r1, r2, r3, sem, s1, s2, o0, o1, o2, o3):
    wid = lax.axis_index("s") * NC + lax.axis_index("c")
    ebase = wid * BPW

    pltpu.async_copy(a0_hbm.at[pl.ds(ebase, BPW)], a0_v, sem)
    pltpu.async_copy(a1_hbm.at[pl.ds(ebase, BPW)], a1_v, s1)
    pltpu.async_copy(a2_hbm.at[pl.ds(ebase, BPW)], a2_v, s2)
    pltpu.sync_copy(w0_hbm, w0_v)
    pltpu.sync_copy(w1_hbm, w1_v)
    pltpu.sync_copy(w2_hbm, w2_v)

    def t12_row(r, _):
        j = r // V
        k = r % V
        for c in range(HG):
            t12_v[r, pl.ds(c * 16, 16)] = (
                w1_v[j, pl.ds(c * 16, 16)] + w2_v[k, pl.ds(c * 16, 16)]
            )
        return 0

    lax.fori_loop(0, V * V, t12_row, 0)

    def tbl_row(r, _):
        i = r // (V * V)
        m = r % (V * V)
        for c in range(HG):
            tbl_v[r, pl.ds(c * 16, 16)] = (
                w0_v[i, pl.ds(c * 16, 16)] + t12_v[m, pl.ds(c * 16, 16)]
            )
        return 0

    lax.fori_loop(0, NT, tbl_row, 0)

    pltpu.make_async_copy(a0_hbm.at[pl.ds(0, BPW)], a0_v, sem).wait()
    pltpu.make_async_copy(a1_hbm.at[pl.ds(0, BPW)], a1_v, s1).wait()
    pltpu.make_async_copy(a2_hbm.at[pl.ds(0, BPW)], a2_v, s2).wait()

    def idx_group(g, _):
        b = g * 16
        a0 = a0_v[pl.ds(b, 16)]
        a1 = a1_v[pl.ds(b, 16)]
        a2 = a2_v[pl.ds(b, 16)]
        idx_v[g // GRP, pl.ds((g % GRP) * 16, 16)] = a0 * 36 + a1 * 6 + a2
        return 0

    lax.fori_loop(0, NCHUNK * GRP, idx_group, 0)

    def expand(k, rv):
        def group(g, _):
            idx16 = idx_v[k, pl.ds(g * 16, 16)]
            e0 = g * 16
            for j in range(16):
                s = idx16[j]
                row = [tbl_v[s, pl.ds(c * 16, 16)] for c in range(HG)]
                for c in range(HG):
                    rv[e0 + j, pl.ds(c * 16, 16)] = row[c]
            return 0

        lax.fori_loop(0, GRP, group, 0)

    def out_start(k, rv, osem):
        return pltpu.async_copy(rv, out_hbm.at[pl.ds(ebase + k * CHUNK, CHUNK)], osem)

    def out_wait(rv, osem):
        pltpu.make_async_copy(rv, out_hbm.at[pl.ds(0, CHUNK)], osem).wait()

    bufs = (r0, r1, r2, r3)
    sems = (o0, o1, o2, o3)

    for b in range(NBUF):
        expand(b, bufs[b])
        out_start(b, bufs[b], sems[b])

    def ring(i, _):
        for b in range(NBUF):
            k = NBUF + NBUF * i + b
            out_wait(bufs[b], sems[b])
            expand(k, bufs[b])
            out_start(k, bufs[b], sems[b])
        return 0

    lax.fori_loop(0, (NCHUNK - NBUF) // NBUF, ring, 0)

    out_wait(bufs[0], sems[0])
    expand(NCHUNK - 1, bufs[0])
    out_start(NCHUNK - 1, bufs[0], sems[0])

    for b in range(NBUF):
        out_wait(bufs[b], sems[b])


@functools.partial(jax.jit, donate_argnums=())
def kernel(edge_attr, W0, W1, W2):
    ea = edge_attr.astype(jnp.int32)
    a0 = ea[:, 0]
    a1 = ea[:, 1]
    a2 = ea[:, 2]

    sc = pl.kernel(
        _sc_body,
        out_type=jax.ShapeDtypeStruct((E, H), jnp.float32),
        mesh=plsc.VectorSubcoreMesh(core_axis_name="c", subcore_axis_name="s"),
        compiler_params=pltpu.CompilerParams(needs_layout_passes=False),
        scratch_types=[
            pltpu.VMEM((BPW,), jnp.int32),
            pltpu.VMEM((BPW,), jnp.int32),
            pltpu.VMEM((BPW,), jnp.int32),
            pltpu.VMEM((NCHUNK, CHUNK), jnp.int32),
            pltpu.VMEM((V, H), jnp.float32),
            pltpu.VMEM((V, H), jnp.float32),
            pltpu.VMEM((V, H), jnp.float32),
            pltpu.VMEM((V * V, H), jnp.float32),
            pltpu.VMEM((NT, H), jnp.float32),
            pltpu.VMEM((CHUNK, H), jnp.float32),
            pltpu.VMEM((CHUNK, H), jnp.float32),
            pltpu.VMEM((CHUNK, H), jnp.float32),
            pltpu.VMEM((CHUNK, H), jnp.float32),
            pltpu.SemaphoreType.DMA,
            pltpu.SemaphoreType.DMA,
            pltpu.SemaphoreType.DMA,
            pltpu.SemaphoreType.DMA,
            pltpu.SemaphoreType.DMA,
            pltpu.SemaphoreType.DMA,
            pltpu.SemaphoreType.DMA,
        ],
    )
    return sc(a0, a1, a2, W0, W1, W2)


# speedup vs baseline: 13.2362x; 1.0134x over previous
import functools

import jax
import jax.numpy as jnp
from jax import lax
from jax.experimental import pallas as pl
from jax.experimental.pallas import tpu as pltpu
from jax.experimental.pallas import tpu_sc as plsc

E = 320000
H = 128
HG = H // 16
V = 6
NT = V * V * V
NC = 2
NS = 16
NW = NC * NS
BPW = E // NW
CHUNK = 80
NCHUNK = BPW // CHUNK
GRP = CHUNK // 16
NPAIR = (NCHUNK - 1) // 2


def _sc_body(a0_hbm, a1_hbm, a2_hbm, w0_hbm, w1_hbm, w2_hbm, out_hbm,
             a0_v, a1_v, a2_v, idx_v, w0_v, w1_v, w2_v, t12_v, tbl_v,
             r0, r1, sem, s1, s2, o0, o1):
    wid = lax.axis_index("s") * NC + lax.axis_index("c")
    ebase = wid * BPW

    pltpu.async_copy(a0_hbm.at[pl.ds(ebase, BPW)], a0_v, sem)
    pltpu.async_copy(a1_hbm.at[pl.ds(ebase, BPW)], a1_v, s1)
    pltpu.async_copy(a2_hbm.at[pl.ds(ebase, BPW)], a2_v, s2)
    pltpu.sync_copy(w0_hbm, w0_v)
    pltpu.sync_copy(w1_hbm, w1_v)
    pltpu.sync_copy(w2_hbm, w2_v)

    def t12_row(r, _):
        j = r // V
        k = r % V
        for c in range(HG):
            t12_v[r, pl.ds(c * 16, 16)] = (
                w1_v[j, pl.ds(c * 16, 16)] + w2_v[k, pl.ds(c * 16, 16)]
            )
        return 0

    lax.fori_loop(0, V * V, t12_row, 0)

    def tbl_row(r, _):
        i = r // (V * V)
        m = r % (V * V)
        for c in range(HG):
            tbl_v[r, pl.ds(c * 16, 16)] = (
                w0_v[i, pl.ds(c * 16, 16)] + t12_v[m, pl.ds(c * 16, 16)]
            )
        return 0

    lax.fori_loop(0, NT, tbl_row, 0)

    pltpu.make_async_copy(a0_hbm.at[pl.ds(0, BPW)], a0_v, sem).wait()
    pltpu.make_async_copy(a1_hbm.at[pl.ds(0, BPW)], a1_v, s1).wait()
    pltpu.make_async_copy(a2_hbm.at[pl.ds(0, BPW)], a2_v, s2).wait()

    def idx_group(g, _):
        b = g * 16
        a0 = a0_v[pl.ds(b, 16)]
        a1 = a1_v[pl.ds(b, 16)]
        a2 = a2_v[pl.ds(b, 16)]
        idx_v[g // GRP, pl.ds((g % GRP) * 16, 16)] = a0 * 36 + a1 * 6 + a2
        return 0

    lax.fori_loop(0, NCHUNK * GRP, idx_group, 0)

    def expand(k, rv):
        def group(g, _):
            idx16 = idx_v[k, pl.ds(g * 16, 16)]
            e0 = g * 16
            for j in range(16):
                s = idx16[j]
                row = [tbl_v[s, pl.ds(c * 16, 16)] for c in range(HG)]
                for c in range(HG):
                    rv[e0 + j, pl.ds(c * 16, 16)] = row[c]
            return 0

        lax.fori_loop(0, GRP, group, 0)

    def out_start(k, rv, osem):
        return pltpu.async_copy(rv, out_hbm.at[pl.ds(ebase + k * CHUNK, CHUNK)], osem)

    def out_wait(rv, osem):
        pltpu.make_async_copy(rv, out_hbm.at[pl.ds(0, CHUNK)], osem).wait()

    expand(0, r0)
    out_start(0, r0, o0)

    def pair(i, _):
        a = 2 * i + 1
        b = 2 * i + 2

        @pl.when(i > 0)
        def _():
            out_wait(r1, o1)

        expand(a, r1)
        out_start(a, r1, o1)

        out_wait(r0, o0)
        expand(b, r0)
        out_start(b, r0, o0)
        return 0

    lax.fori_loop(0, NPAIR, pair, 0)
    out_wait(r1, o1)
    out_wait(r0, o0)


@functools.partial(jax.jit, donate_argnums=())
def kernel(edge_attr, W0, W1, W2):
    ea = edge_attr.astype(jnp.int32)
    a0 = ea[:, 0]
    a1 = ea[:, 1]
    a2 = ea[:, 2]

    sc = pl.kernel(
        _sc_body,
        out_type=jax.ShapeDtypeStruct((E, H), jnp.float32),
        mesh=plsc.VectorSubcoreMesh(core_axis_name="c", subcore_axis_name="s"),
        compiler_params=pltpu.CompilerParams(needs_layout_passes=False),
        scratch_types=[
            pltpu.VMEM((BPW,), jnp.int32),
            pltpu.VMEM((BPW,), jnp.int32),
            pltpu.VMEM((BPW,), jnp.int32),
            pltpu.VMEM((NCHUNK, CHUNK), jnp.int32),
            pltpu.VMEM((V, H), jnp.float32),
            pltpu.VMEM((V, H), jnp.float32),
            pltpu.VMEM((V, H), jnp.float32),
            pltpu.VMEM((V * V, H), jnp.float32),
            pltpu.VMEM((NT, H), jnp.float32),
            pltpu.VMEM((CHUNK, H), jnp.float32),
            pltpu.VMEM((CHUNK, H), jnp.float32),
            pltpu.SemaphoreType.DMA,
            pltpu.SemaphoreType.DMA,
            pltpu.SemaphoreType.DMA,
            pltpu.SemaphoreType.DMA,
            pltpu.SemaphoreType.DMA,
        ],
    )
    return sc(a0, a1, a2, W0, W1, W2)
